# baseline (speedup 1.0000x reference)
import functools

import jax
import jax.numpy as jnp
from jax import lax
from jax.experimental import pallas as pl
from jax.experimental.pallas import tpu as pltpu
from jax.experimental.pallas import tpu_sc as plsc

_NH = 4 * 512
_F = 0
_HB = 128
_NC, _NS = 2, 16
_NW = _NC * _NS
_RPS = 4
_RPW = (_NH - _F) // _NW
_STEPS = _RPW // _RPS
_HALF = _STEPS // 2


def _to_view_colors(colors):
    n, h, w, k, c = colors.shape
    t = colors.transpose(0, 1, 4, 3, 2)
    t = t.reshape(n, h, c, k, w // 128, 128)
    t = t.transpose(0, 1, 2, 4, 3, 5)
    return t.reshape(n * h, c, w // 128, k, 128)


def _to_view_pix(pix):
    n, h, w, k = pix.shape
    t = pix.transpose(0, 1, 3, 2)
    t = t.reshape(n, h, k, w // 128, 128)
    t = t.transpose(0, 1, 3, 2, 4)
    return t.reshape(n * h, w // 128, k, 128)


def _from_view_out(out, n, h, w):
    t = out.reshape(n, h, 4, 4, 128)
    t = t.transpose(0, 1, 2, 4, 3)
    return t.reshape(n, h, w, 4)


def _tc_body(colors_ref, pix_ref, out_ref):
    cb = colors_ref[...]
    pb = pix_ref[...]
    c0 = cb.reshape(_HB, 3, 2, 2, 4, 128)[:, :, :, :, 0, :]
    p0 = pb.reshape(_HB, 2, 2, 4, 128)[:, :, :, 0, :]
    bg = p0 < 0
    one = jnp.float32(1.0)
    r = jnp.where(bg, one, c0[:, 0])
    g = jnp.where(bg, one, c0[:, 1])
    b = jnp.where(bg, one, c0[:, 2])
    a = jnp.where(bg, jnp.float32(0.0), one)
    out = jnp.stack([r, g, b, a], axis=-2)
    out_ref[...] = out.reshape(_HB, 2, 8, 128)


def _tc_half(colors_v, pix_v):
    colors_t = colors_v.reshape(_NH, 3, 2, 8, 128)
    pix_t = pix_v.reshape(_NH, 2, 8, 128)
    out = pl.pallas_call(
        _tc_body,
        grid=(_F // _HB,),
        in_specs=[
            pl.BlockSpec((_HB, 3, 2, 8, 128), lambda i: (i, 0, 0, 0, 0)),
            pl.BlockSpec((_HB, 2, 8, 128), lambda i: (i, 0, 0, 0)),
        ],
        out_specs=pl.BlockSpec((_HB, 2, 8, 128), lambda i: (i, 0, 0, 0)),
        out_shape=jax.ShapeDtypeStruct((_F, 2, 8, 128), jnp.float32),
    )(colors_t, pix_t)
    return out.reshape(_F, 4, 4, 128)


def _sc_shader(colors_hbm, pix_hbm, out_hbm, stage_c, stage_p, stage_o,
               sem_c0, sem_c1, sem_p0, sem_p1, sem_o0, sem_o1):
    wid = lax.axis_index("s") * _NC + lax.axis_index("c")
    base = wid * _RPW
    sems = ((sem_c0, sem_p0, sem_o0), (sem_c1, sem_p1, sem_o1))

    def in_copies(step_idx, b):
        nh0 = _F + base + step_idx * _RPS
        sc, sp, _ = sems[b]
        hrps = _RPS // 2
        return (
            pltpu.make_async_copy(
                colors_hbm.at[pl.ds(nh0, hrps), :, :, 0, :],
                stage_c.at[b, pl.ds(0, hrps)], sc),
            pltpu.make_async_copy(
                colors_hbm.at[pl.ds(nh0 + hrps, hrps), :, :, 0, :],
                stage_c.at[b, pl.ds(hrps, hrps)], sc),
            pltpu.make_async_copy(
                pix_hbm.at[pl.ds(nh0, _RPS)], stage_p.at[b], sp),
        )

    def out_copy(step_idx, b):
        nh0 = base + step_idx * _RPS
        return pltpu.make_async_copy(
            stage_o.at[b], out_hbm.at[pl.ds(nh0, _RPS)], sems[b][2])

    def compute(b):
        one = jnp.float32(1.0)
        zero = jnp.float32(0.0)
        for rr in range(_RPS):
            for wt in range(4):
                for g in range(8):
                    sl = pl.ds(g * 16, 16)
                    bg = stage_p[b, rr, wt, 0, sl] < 0
                    for c in range(3):
                        stage_o[b, rr, wt, c, sl] = jnp.where(
                            bg, one, stage_c[b, rr, c, wt, sl])
                    stage_o[b, rr, wt, 3, sl] = jnp.where(bg, zero, one)

    def handle(step_idx, b, j):
        for cp in in_copies(step_idx, b):
            cp.wait()
        @pl.when(j > 0)
        def _():
            out_copy(step_idx - 2, b).wait()
        compute(b)
        out_copy(step_idx, b).start()

    for cp in in_copies(0, 0):
        cp.start()

    def body(j, carry):
        i0 = 2 * j
        i1 = 2 * j + 1
        for cp in in_copies(i1, 1):
            cp.start()
        handle(i0, 0, j)

        @pl.when(j + 1 < _HALF)
        def _():
            for cp in in_copies(i0 + 2, 0):
                cp.start()
        handle(i1, 1, j)
        return carry

    lax.fori_loop(0, _HALF, body, jnp.int32(0))
    out_copy(_STEPS - 2, 0).wait()
    out_copy(_STEPS - 1, 1).wait()


def _sc_half(colors_v, pix_v):
    mesh = plsc.VectorSubcoreMesh(core_axis_name="c", subcore_axis_name="s")
    sc_call = functools.partial(
        pl.kernel,
        mesh=mesh,
        out_type=jax.ShapeDtypeStruct((_NH - _F, 4, 4, 128), jnp.float32),
        scratch_types=[
            pltpu.VMEM((2, _RPS, 3, 4, 128), jnp.float32),
            pltpu.VMEM((2, _RPS, 4, 4, 128), jnp.int32),
            pltpu.VMEM((2, _RPS, 4, 4, 128), jnp.float32),
            pltpu.SemaphoreType.DMA,
            pltpu.SemaphoreType.DMA,
            pltpu.SemaphoreType.DMA,
            pltpu.SemaphoreType.DMA,
            pltpu.SemaphoreType.DMA,
            pltpu.SemaphoreType.DMA,
        ],
    )(_sc_shader)
    return sc_call(colors_v, pix_v)


def kernel(colors, pix_to_face):
    n, h, w = colors.shape[0], colors.shape[1], colors.shape[2]
    colors_v = _to_view_colors(colors)
    pix_v = _to_view_pix(pix_to_face)
    out = _sc_half(colors_v, pix_v)
    return _from_view_out(out, n, h, w)


# speedup vs baseline: 1.0962x; 1.0962x over previous
import functools

import jax
import jax.numpy as jnp
from jax import lax
from jax.experimental import pallas as pl
from jax.experimental.pallas import tpu as pltpu
from jax.experimental.pallas import tpu_sc as plsc

_NH = 4 * 512
_NC, _NS = 2, 16
_NW = _NC * _NS
_RPS = 4
_RPW = _NH // _NW
_STEPS = _RPW // _RPS
_HALF = _STEPS // 2


def _to_view_colors(colors):
    n, h, w, k, c = colors.shape
    t = colors.transpose(0, 1, 4, 3, 2)
    t = t.reshape(n, h, c, k, w // 128, 128)
    t = t.transpose(0, 1, 2, 4, 3, 5)
    return t.reshape(n * h, c, w // 128, k, 128)


def _to_view_pix(pix):
    n, h, w, k = pix.shape
    t = pix.transpose(0, 1, 3, 2)
    t = t.reshape(n, h, k, w // 128, 128)
    t = t.transpose(0, 1, 3, 2, 4)
    return t.reshape(n * h, w // 128, k, 128)


def _from_view_out(out, n, h, w):
    t = out.reshape(n, h, 4, 4, 128)
    t = t.transpose(0, 1, 2, 4, 3)
    return t.reshape(n, h, w, 4)


def _sc_shader(colors_hbm, pix_hbm, out_hbm, stage_c, stage_p, stage_o,
               sem_c0, sem_c1, sem_p0, sem_p1, sem_o0, sem_o1):
    wid = lax.axis_index("s") * _NC + lax.axis_index("c")
    base = wid * _RPW
    sems = ((sem_c0, sem_p0, sem_o0), (sem_c1, sem_p1, sem_o1))

    def in_copies(step_idx, b):
        nh0 = base + step_idx * _RPS
        sc, sp, _ = sems[b]
        hrps = _RPS // 2
        return (
            pltpu.make_async_copy(
                colors_hbm.at[pl.ds(nh0, hrps), :, :, 0, :],
                stage_c.at[b, pl.ds(0, hrps)], sc),
            pltpu.make_async_copy(
                colors_hbm.at[pl.ds(nh0 + hrps, hrps), :, :, 0, :],
                stage_c.at[b, pl.ds(hrps, hrps)], sc),
            pltpu.make_async_copy(
                pix_hbm.at[pl.ds(nh0, _RPS), :, 0, :], stage_p.at[b], sp),
        )

    def out_copy(step_idx, b):
        nh0 = base + step_idx * _RPS
        return pltpu.make_async_copy(
            stage_o.at[b], out_hbm.at[pl.ds(nh0, _RPS)], sems[b][2])

    def compute(b):
        one = jnp.float32(1.0)
        zero = jnp.float32(0.0)
        for rr in range(_RPS):
            for wt in range(4):
                for g in range(8):
                    sl = pl.ds(g * 16, 16)
                    bg = stage_p[b, rr, wt, sl] < 0
                    for c in range(3):
                        stage_o[b, rr, wt, c, sl] = jnp.where(
                            bg, one, stage_c[b, rr, c, wt, sl])
                    stage_o[b, rr, wt, 3, sl] = jnp.where(bg, zero, one)

    def handle(step_idx, b, j):
        for cp in in_copies(step_idx, b):
            cp.wait()
        @pl.when(j > 0)
        def _():
            out_copy(step_idx - 2, b).wait()
        compute(b)
        out_copy(step_idx, b).start()

    for cp in in_copies(0, 0):
        cp.start()

    def body(j, carry):
        i0 = 2 * j
        i1 = 2 * j + 1
        for cp in in_copies(i1, 1):
            cp.start()
        handle(i0, 0, j)

        @pl.when(j + 1 < _HALF)
        def _():
            for cp in in_copies(i0 + 2, 0):
                cp.start()
        handle(i1, 1, j)
        return carry

    lax.fori_loop(0, _HALF, body, jnp.int32(0))
    out_copy(_STEPS - 2, 0).wait()
    out_copy(_STEPS - 1, 1).wait()


def kernel(colors, pix_to_face):
    n, h, w = colors.shape[0], colors.shape[1], colors.shape[2]
    colors_v = _to_view_colors(colors)
    pix_v = _to_view_pix(pix_to_face)
    mesh = plsc.VectorSubcoreMesh(core_axis_name="c", subcore_axis_name="s")
    sc_call = functools.partial(
        pl.kernel,
        mesh=mesh,
        out_type=jax.ShapeDtypeStruct((_NH, 4, 4, 128), jnp.float32),
        scratch_types=[
            pltpu.VMEM((2, _RPS, 3, 4, 128), jnp.float32),
            pltpu.VMEM((2, _RPS, 4, 128), jnp.int32),
            pltpu.VMEM((2, _RPS, 4, 4, 128), jnp.float32),
            pltpu.SemaphoreType.DMA,
            pltpu.SemaphoreType.DMA,
            pltpu.SemaphoreType.DMA,
            pltpu.SemaphoreType.DMA,
            pltpu.SemaphoreType.DMA,
            pltpu.SemaphoreType.DMA,
        ],
    )(_sc_shader)
    out = sc_call(colors_v, pix_v)
    return _from_view_out(out, n, h, w)
